# Initial kernel scaffold; baseline (speedup 1.0000x reference)
import jax
import jax.numpy as jnp
from jax.experimental import pallas as pl


def kernel(x, edge_index, N, M, W_in, b_in, gcn_W, gcn_b, bn_gamma, bn_beta, gat_W, att_src, att_dst, gat_b, gbn_gamma, gbn_beta, W_out, b_out):
    raise NotImplementedError("write your pallas kernel here")


# speedup vs baseline: 3.0458x; 3.0458x over previous
import functools
import jax
import jax.numpy as jnp
from jax import lax
from jax.experimental import pallas as pl
from jax.experimental.pallas import tpu as pltpu
from jax.experimental.pallas import tpu_sc as plsc

N_NODES = 50000
NPAD = 50048
HID = 256
N_GCN = 6
HEADS = 4
E_EDGES = 65536
EPS = 1e-5

SUBCORES = 16
CORES = 2
FB = 32
BATCH = 512
EPT = E_EDGES // SUBCORES
NBAT = EPT // BATCH
STRIPE = NPAD // SUBCORES
ZROWS = 136

_mesh = plsc.VectorSubcoreMesh(core_axis_name="c", subcore_axis_name="s")
_sc_params = pltpu.CompilerParams(use_tc_tiling_on_sc=False)


def _f32(*shape):
    return jax.ShapeDtypeStruct(shape, jnp.float32)


def _sc_seg_call(table2d, sidxc, didx2, zrows_hbm, nb_total):
    nbc = nb_total // CORES if nb_total > 1 else 1

    @functools.partial(
        pl.kernel,
        out_type=_f32(nb_total * NPAD, FB),
        mesh=_mesh,
        compiler_params=_sc_params,
        scratch_types=[
            pltpu.VMEM((BATCH,), jnp.int32),
            pltpu.VMEM((BATCH,), jnp.int32),
            pltpu.VMEM((BATCH, FB), jnp.float32),
            pltpu.VMEM((ZROWS, FB), jnp.float32),
            pltpu.VMEM_SHARED((NPAD, FB), jnp.float32),
            pltpu.SemaphoreType.DMA,
        ],
    )
    def k(table_h, sidx_h, didx_h, zrows_h, out_h, sidx_v, didx_v, buf_v,
          zbuf_v, acc, sem):
        c = lax.axis_index("c")
        s = lax.axis_index("s")
        pltpu.sync_copy(zrows_h, zbuf_v)
        for j in range(nbc):
            if nb_total > 1:
                ck = c * nbc + j
            else:
                ck = s * 0
            for z in range(STRIPE // ZROWS):
                pltpu.sync_copy(
                    zbuf_v, acc.at[pl.ds(s * STRIPE + z * ZROWS, ZROWS)])
            plsc.subcore_barrier()
            for b in range(NBAT):
                pltpu.sync_copy(
                    sidx_h.at[(ck * SUBCORES + s) * NBAT + b], sidx_v)
                pltpu.sync_copy(didx_h.at[s * NBAT + b], didx_v)
                pltpu.async_copy(table_h.at[sidx_v], buf_v, sem).wait()
                pltpu.sync_copy(buf_v, acc.at[didx_v], add=True)
            plsc.subcore_barrier()
            if nb_total > 1:
                pltpu.sync_copy(
                    acc.at[pl.ds(s * STRIPE, STRIPE)],
                    out_h.at[pl.ds(ck * NPAD + s * STRIPE, STRIPE)])
            else:
                @pl.when(c == 0)
                def _():
                    pltpu.sync_copy(
                        acc.at[pl.ds(s * STRIPE, STRIPE)],
                        out_h.at[pl.ds(s * STRIPE, STRIPE)])

    return k(table2d, sidxc, didx2, zrows_hbm)


def _sc_seg_scaled_call(table2d, wexp3, sidxc, didx2, zrows_hbm):
    nb_total = HEADS * (HID // FB)
    nbc = nb_total // CORES
    fb_per_head = HID // FB

    @functools.partial(
        pl.kernel,
        out_type=_f32(nb_total * NPAD, FB),
        mesh=_mesh,
        compiler_params=_sc_params,
        scratch_types=[
            pltpu.VMEM((BATCH,), jnp.int32),
            pltpu.VMEM((BATCH,), jnp.int32),
            pltpu.VMEM((BATCH, FB), jnp.float32),
            pltpu.VMEM((BATCH, 16), jnp.float32),
            pltpu.VMEM((ZROWS, FB), jnp.float32),
            pltpu.VMEM_SHARED((NPAD, FB), jnp.float32),
            pltpu.SemaphoreType.DMA,
        ],
    )
    def k(table_h, wexp_h, sidx_h, didx_h, zrows_h, out_h, sidx_v, didx_v,
          buf_v, w_v, zbuf_v, acc, sem):
        c = lax.axis_index("c")
        s = lax.axis_index("s")
        pltpu.sync_copy(zrows_h, zbuf_v)

        def chunk_body(j, carry):
            ck = c * nbc + j
            head = ck // fb_per_head
            for z in range(STRIPE // ZROWS):
                pltpu.sync_copy(
                    zbuf_v, acc.at[pl.ds(s * STRIPE + z * ZROWS, ZROWS)])
            plsc.subcore_barrier()
            for b in range(NBAT):
                pltpu.sync_copy(
                    sidx_h.at[(ck * SUBCORES + s) * NBAT + b], sidx_v)
                pltpu.sync_copy(didx_h.at[s * NBAT + b], didx_v)
                pltpu.async_copy(table_h.at[sidx_v], buf_v, sem).wait()
                pltpu.sync_copy(
                    wexp_h.at[(head * SUBCORES + s) * NBAT + b], w_v)

                def scale_body(e, carry2):
                    w = w_v[e, :]
                    buf_v[e, pl.ds(0, 16)] = buf_v[e, pl.ds(0, 16)] * w
                    buf_v[e, pl.ds(16, 16)] = buf_v[e, pl.ds(16, 16)] * w
                    return carry2

                lax.fori_loop(0, BATCH, scale_body, 0)
                pltpu.sync_copy(buf_v, acc.at[didx_v], add=True)
            plsc.subcore_barrier()
            pltpu.sync_copy(
                acc.at[pl.ds(s * STRIPE, STRIPE)],
                out_h.at[pl.ds(ck * NPAD + s * STRIPE, STRIPE)])
            return carry

        lax.fori_loop(0, nbc, chunk_body, 0)

    return k(table2d, wexp3, sidxc, didx2, zrows_hbm)


def _sc_gat_w_call(src_t, dsta, dstm, mask4_hbm, sidx2, didx2, zrows_hbm):
    @functools.partial(
        pl.kernel,
        out_type=(_f32(E_EDGES, 16), _f32(NPAD, 16)),
        mesh=_mesh,
        compiler_params=_sc_params,
        scratch_types=[
            pltpu.VMEM((BATCH,), jnp.int32),
            pltpu.VMEM((BATCH,), jnp.int32),
            pltpu.VMEM((BATCH, 16), jnp.float32),
            pltpu.VMEM((BATCH, 16), jnp.float32),
            pltpu.VMEM((BATCH, 16), jnp.float32),
            pltpu.VMEM((BATCH, 16), jnp.float32),
            pltpu.VMEM((16,), jnp.float32),
            pltpu.VMEM((ZROWS, 16), jnp.float32),
            pltpu.VMEM_SHARED((NPAD, 16), jnp.float32),
            pltpu.SemaphoreType.DMA,
        ],
    )
    def k(srct_h, dsta_h, dstm_h, mask_h, sidx_h, didx_h, zrows_h, wout_h,
          den_h, sidx_v, didx_v, sa_v, da_v, dm_v, w_v, m_v, zbuf_v, acc,
          sem):
        c = lax.axis_index("c")
        s = lax.axis_index("s")

        @pl.when(c == 0)
        def _():
            pltpu.sync_copy(mask_h, m_v)
            pltpu.sync_copy(zrows_h, zbuf_v)
            for z in range(STRIPE // ZROWS):
                pltpu.sync_copy(
                    zbuf_v, acc.at[pl.ds(s * STRIPE + z * ZROWS, ZROWS)])
            plsc.subcore_barrier()
            for b in range(NBAT):
                pltpu.sync_copy(sidx_h.at[s * NBAT + b], sidx_v)
                pltpu.sync_copy(didx_h.at[s * NBAT + b], didx_v)
                pltpu.async_copy(srct_h.at[sidx_v], sa_v, sem).wait()
                pltpu.async_copy(dsta_h.at[didx_v], da_v, sem).wait()
                pltpu.async_copy(dstm_h.at[didx_v], dm_v, sem).wait()

                def wbody(e, carry):
                    t = sa_v[e, :] + da_v[e, :]
                    t = jnp.maximum(t, 0.2 * t)
                    w_v[e, :] = jnp.exp(t - dm_v[e, :]) * m_v[:]
                    return carry

                lax.fori_loop(0, BATCH, wbody, 0)
                pltpu.sync_copy(
                    w_v, wout_h.at[pl.ds(s * EPT + b * BATCH, BATCH)])
                pltpu.sync_copy(w_v, acc.at[didx_v], add=True)
            plsc.subcore_barrier()
            pltpu.sync_copy(
                acc.at[pl.ds(s * STRIPE, STRIPE)],
                den_h.at[pl.ds(s * STRIPE, STRIPE)])

    return k(src_t, dsta, dstm, mask4_hbm, sidx2, didx2, zrows_hbm)


def _sc_edge_score_call(s1t, s2t, sidx2, didx2):
    ept = E_EDGES // (SUBCORES * CORES)
    nbat = ept // BATCH

    @functools.partial(
        pl.kernel,
        out_type=_f32(E_EDGES, 16),
        mesh=_mesh,
        compiler_params=_sc_params,
        scratch_types=[
            pltpu.VMEM((BATCH,), jnp.int32),
            pltpu.VMEM((BATCH,), jnp.int32),
            pltpu.VMEM((BATCH, 16), jnp.float32),
            pltpu.VMEM((BATCH, 16), jnp.float32),
            pltpu.SemaphoreType.DMA,
        ],
    )
    def k(s1_h, s2_h, sidx_h, didx_h, out_h, sidx_v, didx_v, a_v, b_v, sem):
        c = lax.axis_index("c")
        s = lax.axis_index("s")
        w = s * CORES + c
        for b in range(nbat):
            pltpu.sync_copy(sidx_h.at[w * nbat + b], sidx_v)
            pltpu.sync_copy(didx_h.at[w * nbat + b], didx_v)
            pltpu.async_copy(s1_h.at[sidx_v], a_v, sem).wait()
            pltpu.async_copy(s2_h.at[didx_v], b_v, sem).wait()

            def abody(e, carry):
                a_v[e, :] = a_v[e, :] + b_v[e, :]
                return carry

            lax.fori_loop(0, BATCH, abody, 0)
            pltpu.sync_copy(
                a_v, out_h.at[pl.ds(w * ept + b * BATCH, BATCH)])

    return k(s1t, s2t, sidx2, didx2)


RB = 1088
GRID = NPAD // RB


def _full(shape):
    return pl.BlockSpec(shape, lambda i: (0,) * len(shape))


def _rows(shape):
    return pl.BlockSpec(shape, lambda i: (i,) + (0,) * (len(shape) - 1))


def _k0_body(x_ref, w_ref, b_ref, deg_ref, h_ref, g_ref, dinv_ref):
    h = jnp.maximum(jnp.dot(x_ref[...], w_ref[...]) + b_ref[...], 0.0)
    dinv = lax.rsqrt(deg_ref[:, 0:1] + 1.0)
    h_ref[...] = h
    g_ref[...] = dinv * h
    dinv_ref[...] = jnp.broadcast_to(dinv, (RB, 8))


def _tc_input(x_pad, w_in_pad, b_in, deg32):
    return pl.pallas_call(
        _k0_body,
        grid=(GRID,),
        in_specs=[_rows((RB, 128)), _full((128, HID)), _full((1, HID)),
                  _rows((RB, FB))],
        out_specs=[_rows((RB, HID)), _rows((RB, HID)), _rows((RB, 8))],
        out_shape=[_f32(NPAD, HID), _f32(NPAD, HID), _f32(NPAD, 8)],
    )(x_pad, w_in_pad, b_in, deg32)


def _k1_body(agg_ref, g_ref, dinv_ref, w_ref, b_ref, z_ref, ssum_ref,
             ssq_ref):
    i = pl.program_id(0)
    zin = dinv_ref[:, 0:1] * (agg_ref[...] + g_ref[...])
    z = jnp.dot(zin, w_ref[...]) + b_ref[...]
    z_ref[...] = z
    valid = (i * RB + lax.broadcasted_iota(jnp.int32, (RB, 1), 0)) < N_NODES
    z = jnp.where(valid, z, 0.0)

    @pl.when(i == 0)
    def _():
        ssum_ref[...] = jnp.zeros_like(ssum_ref)
        ssq_ref[...] = jnp.zeros_like(ssq_ref)

    ssum_ref[0:1, :] += jnp.sum(z, axis=0, keepdims=True)
    ssq_ref[0:1, :] += jnp.sum(z * z, axis=0, keepdims=True)


def _tc_gcn_mm(aggf, g, dinv8, w, b):
    return pl.pallas_call(
        _k1_body,
        grid=(GRID,),
        in_specs=[_rows((RB, HID)), _rows((RB, HID)), _rows((RB, 8)),
                  _full((HID, HID)), _full((1, HID))],
        out_specs=[_rows((RB, HID)), _full((8, HID)), _full((8, HID))],
        out_shape=[_f32(NPAD, HID), _f32(8, HID), _f32(8, HID)],
    )(aggf, g, dinv8, w, b)


def _k2_body(use_res, z_ref, res_ref, ssum_ref, ssq_ref, gam_ref, bet_ref,
             dinv_ref, h_ref, g_ref):
    n = jnp.float32(N_NODES)
    mean = ssum_ref[0:1, :] / n
    var = ssq_ref[0:1, :] / n - mean * mean
    rstd = lax.rsqrt(var + EPS)
    h = jnp.maximum((z_ref[...] - mean) * rstd * gam_ref[...] + bet_ref[...],
                    0.0)
    if use_res:
        h = h + res_ref[...]
    h_ref[...] = h
    g_ref[...] = dinv_ref[:, 0:1] * h


def _tc_gcn_norm(z, res, ssum, ssq, gamma, beta, dinv8, use_res):
    return pl.pallas_call(
        functools.partial(_k2_body, use_res),
        grid=(GRID,),
        in_specs=[_rows((RB, HID)), _rows((RB, HID)), _full((8, HID)),
                  _full((8, HID)), _full((1, HID)), _full((1, HID)),
                  _rows((RB, 8))],
        out_specs=[_rows((RB, HID)), _rows((RB, HID))],
        out_shape=[_f32(NPAD, HID), _f32(NPAD, HID)],
    )(z, res, ssum, ssq, gamma, beta, dinv8)


def _k3_body(h_ref, w_ref, as_ref, ad_ref, h4_ref, al_ref, gmax_ref):
    i = pl.program_id(0)
    h4 = jnp.dot(h_ref[...], w_ref[...])
    h4_ref[...] = h4
    ps = h4 * as_ref[...]
    pd = h4 * ad_ref[...]
    cols = []
    for hd in range(HEADS):
        cols.append(jnp.sum(ps[:, hd * HID:(hd + 1) * HID], axis=1,
                            keepdims=True))
    for hd in range(HEADS):
        cols.append(jnp.sum(pd[:, hd * HID:(hd + 1) * HID], axis=1,
                            keepdims=True))
    al = jnp.concatenate(cols, axis=1)
    al_ref[...] = al
    valid = (i * RB + lax.broadcasted_iota(jnp.int32, (RB, 1), 0)) < N_NODES
    bmax = jnp.max(jnp.where(valid, al[:, 0:4], -1e30), axis=0,
                   keepdims=True)
    bmax = jnp.concatenate(
        [bmax, jnp.full((1, 124), -1e30, jnp.float32)], axis=1)

    @pl.when(i == 0)
    def _():
        gmax_ref[...] = jnp.full_like(gmax_ref, -1e30)

    gmax_ref[0:1, :] = jnp.maximum(gmax_ref[0:1, :], bmax)


def _tc_gat_pre(h, gat_w, as_flat, ad_flat):
    return pl.pallas_call(
        _k3_body,
        grid=(GRID,),
        in_specs=[_rows((RB, HID)), _full((HID, HEADS * HID)),
                  _full((1, HEADS * HID)), _full((1, HEADS * HID))],
        out_specs=[_rows((RB, HEADS * HID)), _rows((RB, 8)),
                   _full((8, 128))],
        out_shape=[_f32(NPAD, HEADS * HID), _f32(NPAD, 8), _f32(8, 128)],
    )(h, gat_w, as_flat, ad_flat)


def _k4_body(al_ref, gmax_ref, srct_ref, dsta_ref, dstm_ref, wself_ref):
    als = al_ref[:, 0:4]
    ald = al_ref[:, 4:8]
    g4 = gmax_ref[0:1, 0:4]
    t = g4 + ald
    md = jnp.maximum(t, 0.2 * t)
    ts = als + ald
    ts = jnp.maximum(ts, 0.2 * ts)
    wself = jnp.exp(ts - md)
    z12 = jnp.zeros((RB, 12), jnp.float32)
    z4 = jnp.zeros((RB, 4), jnp.float32)
    srct_ref[...] = jnp.concatenate([als, z12], axis=1)
    dsta_ref[...] = jnp.concatenate([ald, z12], axis=1)
    dstm_ref[...] = jnp.concatenate([md, z12], axis=1)
    wself_ref[...] = jnp.concatenate([wself, z4], axis=1)


def _tc_gat_tables(al, gmax):
    return pl.pallas_call(
        _k4_body,
        grid=(GRID,),
        in_specs=[_rows((RB, 8)), _full((8, 128))],
        out_specs=[_rows((RB, 16)), _rows((RB, 16)), _rows((RB, 16)),
                   _rows((RB, 8))],
        out_shape=[_f32(NPAD, 16), _f32(NPAD, 16), _f32(NPAD, 16),
                   _f32(NPAD, 8)],
    )(al, gmax)


def _k6_body(wp_ref, we_ref):
    for hd in range(HEADS):
        col = wp_ref[:, hd:hd + 1]
        we_ref[hd, :, :] = jnp.broadcast_to(col, (2048, 16))


def _tc_wexpand(wpack):
    return pl.pallas_call(
        _k6_body,
        grid=(E_EDGES // 2048,),
        in_specs=[pl.BlockSpec((2048, 16), lambda e: (e, 0))],
        out_specs=pl.BlockSpec((HEADS, 2048, 16), lambda e: (0, e, 0)),
        out_shape=_f32(HEADS, E_EDGES, 16),
    )(wpack)


def _k5a_body(num_ref, den_ref, wself_ref, h4_ref, b_ref, att_ref, ssum_ref,
              ssq_ref):
    i = pl.program_id(0)
    acc = jnp.zeros((RB, HID), jnp.float32)
    for hd in range(HEADS):
        ws = wself_ref[:, hd:hd + 1]
        numh = (num_ref[:, hd * HID:(hd + 1) * HID]
                + ws * h4_ref[:, hd * HID:(hd + 1) * HID])
        denh = den_ref[:, hd:hd + 1] + ws + 1e-16
        acc = acc + numh / denh
    att = acc * (1.0 / HEADS) + b_ref[...]
    att_ref[...] = att
    valid = (i * RB + lax.broadcasted_iota(jnp.int32, (RB, 1), 0)) < N_NODES
    att = jnp.where(valid, att, 0.0)

    @pl.when(i == 0)
    def _():
        ssum_ref[...] = jnp.zeros_like(ssum_ref)
        ssq_ref[...] = jnp.zeros_like(ssq_ref)

    ssum_ref[0:1, :] += jnp.sum(att, axis=0, keepdims=True)
    ssq_ref[0:1, :] += jnp.sum(att * att, axis=0, keepdims=True)


def _tc_gat_post(numf, den16, wself, h4, gat_b):
    return pl.pallas_call(
        _k5a_body,
        grid=(GRID,),
        in_specs=[_rows((RB, HEADS * HID)), _rows((RB, 16)), _rows((RB, 8)),
                  _rows((RB, HEADS * HID)), _full((1, HID))],
        out_specs=[_rows((RB, HID)), _full((8, HID)), _full((8, HID))],
        out_shape=[_f32(NPAD, HID), _f32(8, HID), _f32(8, HID)],
    )(numf, den16, wself, h4, gat_b)


def _k5b_body(att_ref, ssum_ref, ssq_ref, gam_ref, bet_ref, w1_ref, w2_ref,
              bout_ref, s1_ref, s2_ref):
    n = jnp.float32(N_NODES)
    mean = ssum_ref[0:1, :] / n
    var = ssq_ref[0:1, :] / n - mean * mean
    rstd = lax.rsqrt(var + EPS)
    hf = jnp.maximum(
        (att_ref[...] - mean) * rstd * gam_ref[...] + bet_ref[...], 0.0)
    s1 = jnp.sum(hf * w1_ref[...], axis=1, keepdims=True)
    s2 = jnp.sum(hf * w2_ref[...], axis=1, keepdims=True) + bout_ref[0:1, 0:1]
    s1_ref[...] = jnp.broadcast_to(s1, (RB, 16))
    s2_ref[...] = jnp.broadcast_to(s2, (RB, 16))


def _tc_final(att, ssum, ssq, ggam, gbet, w1, w2, bfull):
    return pl.pallas_call(
        _k5b_body,
        grid=(GRID,),
        in_specs=[_rows((RB, HID)), _full((8, HID)), _full((8, HID)),
                  _full((1, HID)), _full((1, HID)), _full((1, HID)),
                  _full((1, HID)), _full((1, HID))],
        out_specs=[_rows((RB, 16)), _rows((RB, 16))],
        out_shape=[_f32(NPAD, 16), _f32(NPAD, 16)],
    )(att, ssum, ssq, ggam, gbet, w1, w2, bfull)


def _block_cols(a, fb=FB):
    n, f = a.shape
    return a.reshape(n, f // fb, fb).transpose(1, 0, 2).reshape(-1, fb)


def _unblock_cols(a2d, f):
    nb = f // FB
    return (a2d.reshape(nb, NPAD, FB).transpose(1, 0, 2).reshape(NPAD, f))


def kernel(x, edge_index, N, M, W_in, b_in, gcn_W, gcn_b, bn_gamma, bn_beta,
           gat_W, att_src, att_dst, gat_b, gbn_gamma, gbn_beta, W_out, b_out):
    src = edge_index[0].astype(jnp.int32)
    dst = edge_index[1].astype(jnp.int32)
    didx2 = dst.reshape(SUBCORES * NBAT, BATCH)
    sidx2 = src.reshape(SUBCORES * NBAT, BATCH)
    sidx2_32 = src.reshape(SUBCORES * CORES * 4, BATCH)
    didx2_32 = dst.reshape(SUBCORES * CORES * 4, BATCH)
    zrows = jnp.zeros((ZROWS, FB), jnp.float32)
    zrows16 = jnp.zeros((ZROWS, 16), jnp.float32)
    mask4 = (jnp.arange(16) < 4).astype(jnp.float32)

    offs8 = (jnp.arange(HID // FB, dtype=jnp.int32) * NPAD)[:, None]
    sidxc8 = (src[None, :] + offs8).reshape((HID // FB) * SUBCORES * NBAT,
                                            BATCH)
    offs32 = (jnp.arange(HEADS * HID // FB, dtype=jnp.int32) * NPAD)[:, None]
    sidxc32 = (src[None, :] + offs32).reshape(
        (HEADS * HID // FB) * SUBCORES * NBAT, BATCH)
    zidxc = jnp.zeros((SUBCORES * NBAT, BATCH), jnp.int32)

    ones_tbl = jnp.ones((8, FB), jnp.float32)
    deg32 = _sc_seg_call(ones_tbl, zidxc, didx2, zrows, 1)

    x_pad = jnp.pad(x, ((0, NPAD - N_NODES), (0, 126)))
    w_in_pad = jnp.pad(W_in, ((0, 126), (0, 0)))
    h, g, dinv8 = _tc_input(x_pad, w_in_pad, b_in.reshape(1, HID), deg32)

    for i in range(N_GCN):
        h_res = h
        gt = _block_cols(g)
        aggt = _sc_seg_call(gt, sidxc8, didx2, zrows, HID // FB)
        aggf = _unblock_cols(aggt, HID)
        z, ssum, ssq = _tc_gcn_mm(aggf, g, dinv8, gcn_W[i],
                                  gcn_b[i].reshape(1, HID))
        h, g = _tc_gcn_norm(z, h_res, ssum, ssq,
                            bn_gamma[i].reshape(1, HID),
                            bn_beta[i].reshape(1, HID), dinv8,
                            use_res=(i > 0 and i % 2 == 1))

    h4, al, gmax = _tc_gat_pre(h, gat_W,
                               att_src.reshape(1, HEADS * HID),
                               att_dst.reshape(1, HEADS * HID))
    src_t, dsta, dstm, wself = _tc_gat_tables(al, gmax)
    wpack, den16 = _sc_gat_w_call(src_t, dsta, dstm, mask4, sidx2, didx2,
                                  zrows16)
    wexp = _tc_wexpand(wpack)
    wexp3 = wexp.reshape(HEADS * SUBCORES * NBAT, BATCH, 16)
    h4t = _block_cols(h4)
    numt = _sc_seg_scaled_call(h4t, wexp3, sidxc32, didx2, zrows)
    numf = _unblock_cols(numt, HEADS * HID)
    att, ssum, ssq = _tc_gat_post(numf, den16, wself, h4,
                                  gat_b.reshape(1, HID))
    bfull = jnp.full((1, HID), b_out[0], jnp.float32)
    s1t, s2t = _tc_final(att, ssum, ssq, gbn_gamma.reshape(1, HID),
                         gbn_beta.reshape(1, HID),
                         W_out[:HID, 0].reshape(1, HID),
                         W_out[HID:, 0].reshape(1, HID), bfull)

    es = _sc_edge_score_call(s1t, s2t, sidx2_32, didx2_32)
    scores = es[:, 0]
    zero_nm = (jnp.asarray(N) * 0 + jnp.asarray(M) * 0).astype(scores.dtype)
    return scores.reshape(256, 256) + zero_nm
